# baseline (speedup 1.0000x reference)
import functools

import jax
import jax.numpy as jnp
from jax import lax
from jax.experimental import pallas as pl
from jax.experimental.pallas import tpu as pltpu
from jax.experimental.pallas import tpu_sc as plsc

N = 10000
E = 320000
D = 128
NW = 32
EPW = E // NW
CHUNK = 128
NCHUNK = 80
EPW_PAD = NCHUNK * CHUNK
NPAD = 10112
RPT = NPAD // 16


def _segsum_body(h_hbm, srcs_hbm, dsts_hbm, zrow_hbm, out_hbm,
                 src_v, dst_v, rows_v, acc, sem):
    cid = lax.axis_index("c")
    sid = lax.axis_index("s")
    wid = sid * 2 + cid

    r0 = sid * RPT
    pltpu.sync_copy(zrow_hbm, acc.at[pl.ds(r0, RPT)])
    pltpu.sync_copy(srcs_hbm.at[wid], src_v)
    pltpu.sync_copy(dsts_hbm.at[wid], dst_v)
    plsc.subcore_barrier()

    def chunk(j, carry):
        pltpu.async_copy(h_hbm.at[src_v.at[j]], rows_v, sem).wait()
        pltpu.sync_copy(rows_v, acc.at[dst_v.at[j]], add=True)
        return carry

    lax.fori_loop(0, NCHUNK, chunk, 0)
    plsc.subcore_barrier()
    pltpu.sync_copy(acc.at[pl.ds(r0, RPT)], out_hbm.at[cid, pl.ds(r0, RPT)])


def _deg_body(dsts_hbm, zrow_hbm, ones_hbm, outdeg_hbm,
              dst_v, ones_v, accd):
    cid = lax.axis_index("c")
    sid = lax.axis_index("s")
    wid = sid * 2 + cid

    r0 = sid * RPT
    pltpu.sync_copy(zrow_hbm, accd.at[pl.ds(r0, RPT)])
    pltpu.sync_copy(ones_hbm, ones_v)
    pltpu.sync_copy(dsts_hbm.at[wid], dst_v)
    plsc.subcore_barrier()

    def chunk(j, carry):
        pltpu.sync_copy(ones_v, accd.at[dst_v.at[j]], add=True)
        return carry

    lax.fori_loop(0, NCHUNK, chunk, 0)
    plsc.subcore_barrier()
    pltpu.sync_copy(accd.at[pl.ds(r0, RPT)],
                    outdeg_hbm.at[cid, pl.ds(r0, RPT)])


def _sc_mesh():
    return plsc.VectorSubcoreMesh(core_axis_name="c", subcore_axis_name="s")


_segsum = pl.kernel(
    _segsum_body, mesh=_sc_mesh(),
    out_type=[jax.ShapeDtypeStruct((2, NPAD, D), jnp.float32)],
    scratch_types=[
        pltpu.VMEM((NCHUNK, CHUNK), jnp.int32),
        pltpu.VMEM((NCHUNK, CHUNK), jnp.int32),
        pltpu.VMEM((CHUNK, D), jnp.float32),
        pltpu.VMEM_SHARED((NPAD, D), jnp.float32),
        pltpu.SemaphoreType.DMA,
    ])

_deg = pl.kernel(
    _deg_body, mesh=_sc_mesh(),
    out_type=[jax.ShapeDtypeStruct((2, NPAD, D), jnp.float32)],
    scratch_types=[
        pltpu.VMEM((NCHUNK, CHUNK), jnp.int32),
        pltpu.VMEM((CHUNK, D), jnp.float32),
        pltpu.VMEM_SHARED((NPAD, D), jnp.float32),
    ])


def _tc_layer_body(first, last, *refs):
    if first:
        (h_ref, p_ref, dp_ref, w1_ref, b1_ref, w2_ref, b2_ref, g_ref, be_ref,
         hn_ref, invd_ref) = refs
    elif last:
        (h_ref, p_ref, invd_in, w1_ref, b1_ref, w2_ref, b2_ref, g_ref, be_ref,
         wp_ref, bp_ref, out_ref) = refs
    else:
        (h_ref, p_ref, invd_in, w1_ref, b1_ref, w2_ref, b2_ref, g_ref, be_ref,
         hn_ref) = refs

    h = h_ref[...]
    p = p_ref[0, :N, :] + p_ref[1, :N, :]
    if first:
        deg = dp_ref[0, :N, 0:1] + dp_ref[1, :N, 0:1]
        invd = 1.0 / jnp.maximum(deg, 1.0)
        invd_ref[...] = invd
    else:
        invd = invd_in[...]
    r = h + p * invd
    z = jnp.maximum(jnp.dot(r, w1_ref[...],
                            preferred_element_type=jnp.float32) + b1_ref[...],
                    0.0)
    z = jnp.dot(z, w2_ref[...], preferred_element_type=jnp.float32) + b2_ref[...]
    mu = jnp.mean(z, axis=0, keepdims=True)
    zc = z - mu
    var = jnp.mean(zc * zc, axis=0, keepdims=True)
    hn = jnp.maximum(zc * (lax.rsqrt(var + 1e-5) * g_ref[...]) + be_ref[...],
                     0.0)
    if last:
        hg = jnp.mean(hn, axis=0, keepdims=True)
        out_ref[...] = jnp.dot(hg, wp_ref[...],
                               preferred_element_type=jnp.float32) + bp_ref[...]
    else:
        hn_ref[...] = hn


def _tc_layer(first, last, args):
    if first:
        out_shape = [jax.ShapeDtypeStruct((N, D), jnp.float32),
                     jax.ShapeDtypeStruct((N, 1), jnp.float32)]
    elif last:
        out_shape = jax.ShapeDtypeStruct((1, D), jnp.float32)
    else:
        out_shape = jax.ShapeDtypeStruct((N, D), jnp.float32)
    return pl.pallas_call(
        functools.partial(_tc_layer_body, first, last),
        out_shape=out_shape,
    )(*args)


def kernel(features, edge_index, W1, b1, W2, b2, gamma, beta, Wp, bp):
    src = edge_index[0].astype(jnp.int32)
    dst = edge_index[1].astype(jnp.int32)
    srcs = jnp.pad(src.reshape(NW, EPW),
                   ((0, 0), (0, EPW_PAD - EPW))).reshape(NW, NCHUNK, CHUNK)
    dsts = jnp.pad(dst.reshape(NW, EPW), ((0, 0), (0, EPW_PAD - EPW)),
                   constant_values=N).reshape(NW, NCHUNK, CHUNK)
    zrow = jnp.zeros((RPT, D), jnp.float32)
    ones = jnp.ones((CHUNK, D), jnp.float32)

    h = features
    invd = None
    (dp,) = _deg(dsts, zrow, ones)
    for i in range(3):
        (p,) = _segsum(h, srcs, dsts, zrow)
        if i == 0:
            h, invd = _tc_layer(True, False,
                                (h, p, dp, W1[i], b1[i], W2[i], b2[i],
                                 gamma[i], beta[i]))
        elif i < 2:
            h = _tc_layer(False, False,
                          (h, p, invd, W1[i], b1[i], W2[i], b2[i],
                           gamma[i], beta[i]))
        else:
            return _tc_layer(False, True,
                             (h, p, invd, W1[i], b1[i], W2[i], b2[i],
                              gamma[i], beta[i], Wp, bp))


# speedup vs baseline: 1.4395x; 1.4395x over previous
import functools

import jax
import jax.numpy as jnp
from jax import lax
from jax.experimental import pallas as pl
from jax.experimental.pallas import tpu as pltpu
from jax.experimental.pallas import tpu_sc as plsc

N = 10000
E = 320000
D = 128
NW = 32
EPW = E // NW
CHUNK = 128
NCHUNK = 79
EPW_PAD = NCHUNK * CHUNK
NPAD = 10112
RPT = NPAD // 16


def _segsum_body(h_hbm, srcs_hbm, dsts_hbm, zrow_hbm, out_hbm,
                 src_v, dst_v, rows_v, acc, sem):
    cid = lax.axis_index("c")
    sid = lax.axis_index("s")
    wid = sid * 2 + cid

    r0 = sid * RPT
    pltpu.sync_copy(zrow_hbm, acc.at[pl.ds(r0, RPT)])
    pltpu.sync_copy(srcs_hbm.at[wid], src_v)
    pltpu.sync_copy(dsts_hbm.at[wid], dst_v)
    plsc.subcore_barrier()

    def chunk(j, carry):
        pltpu.async_copy(h_hbm.at[src_v.at[j]], rows_v, sem).wait()
        pltpu.sync_copy(rows_v, acc.at[dst_v.at[j]], add=True)
        return carry

    lax.fori_loop(0, NCHUNK, chunk, 0)
    plsc.subcore_barrier()
    pltpu.sync_copy(acc.at[pl.ds(r0, RPT)], out_hbm.at[cid, pl.ds(r0, RPT)])


def _deg_body(dsts_hbm, zrow_hbm, ones_hbm, outdeg_hbm,
              dst_v, ones_v, accd):
    cid = lax.axis_index("c")
    sid = lax.axis_index("s")
    wid = sid * 2 + cid

    r0 = sid * RPT
    pltpu.sync_copy(zrow_hbm, accd.at[pl.ds(r0, RPT)])
    pltpu.sync_copy(ones_hbm, ones_v)
    pltpu.sync_copy(dsts_hbm.at[wid], dst_v)
    plsc.subcore_barrier()

    def chunk(j, carry):
        pltpu.sync_copy(ones_v, accd.at[dst_v.at[j]], add=True)
        return carry

    lax.fori_loop(0, NCHUNK, chunk, 0)
    plsc.subcore_barrier()
    pltpu.sync_copy(accd.at[pl.ds(r0, RPT)],
                    outdeg_hbm.at[cid, pl.ds(r0, RPT)])


def _sc_mesh():
    return plsc.VectorSubcoreMesh(core_axis_name="c", subcore_axis_name="s")


_segsum = pl.kernel(
    _segsum_body, mesh=_sc_mesh(),
    out_type=[jax.ShapeDtypeStruct((2, NPAD, D), jnp.float32)],
    scratch_types=[
        pltpu.VMEM((NCHUNK, CHUNK), jnp.int32),
        pltpu.VMEM((NCHUNK, CHUNK), jnp.int32),
        pltpu.VMEM((CHUNK, D), jnp.float32),
        pltpu.VMEM_SHARED((NPAD, D), jnp.float32),
        pltpu.SemaphoreType.DMA,
    ])

_deg = pl.kernel(
    _deg_body, mesh=_sc_mesh(),
    out_type=[jax.ShapeDtypeStruct((2, NPAD, D), jnp.float32)],
    scratch_types=[
        pltpu.VMEM((NCHUNK, CHUNK), jnp.int32),
        pltpu.VMEM((CHUNK, D), jnp.float32),
        pltpu.VMEM_SHARED((NPAD, D), jnp.float32),
    ])


def _tc_layer_body(first, last, *refs):
    if first:
        (h_ref, p_ref, dp_ref, w1_ref, b1_ref, w2_ref, b2_ref, g_ref, be_ref,
         hn_ref, invd_ref) = refs
    elif last:
        (h_ref, p_ref, invd_in, w1_ref, b1_ref, w2_ref, b2_ref, g_ref, be_ref,
         wp_ref, bp_ref, out_ref) = refs
    else:
        (h_ref, p_ref, invd_in, w1_ref, b1_ref, w2_ref, b2_ref, g_ref, be_ref,
         hn_ref) = refs

    h = h_ref[...]
    p = p_ref[0, :N, :] + p_ref[1, :N, :]
    if first:
        deg = dp_ref[0, :N, 0:1] + dp_ref[1, :N, 0:1]
        invd = 1.0 / jnp.maximum(deg, 1.0)
        invd_ref[...] = invd
    else:
        invd = invd_in[...]
    r = h + p * invd
    z = jnp.maximum(jnp.dot(r, w1_ref[...],
                            preferred_element_type=jnp.float32) + b1_ref[...],
                    0.0)
    z = jnp.dot(z, w2_ref[...], preferred_element_type=jnp.float32) + b2_ref[...]
    mu = jnp.mean(z, axis=0, keepdims=True)
    zc = z - mu
    var = jnp.mean(zc * zc, axis=0, keepdims=True)
    hn = jnp.maximum(zc * (lax.rsqrt(var + 1e-5) * g_ref[...]) + be_ref[...],
                     0.0)
    if last:
        hg = jnp.mean(hn, axis=0, keepdims=True)
        out_ref[...] = jnp.dot(hg, wp_ref[...],
                               preferred_element_type=jnp.float32) + bp_ref[...]
    else:
        hn_ref[...] = hn


def _tc_layer(first, last, args):
    if first:
        out_shape = [jax.ShapeDtypeStruct((N, D), jnp.float32),
                     jax.ShapeDtypeStruct((N, 1), jnp.float32)]
    elif last:
        out_shape = jax.ShapeDtypeStruct((1, D), jnp.float32)
    else:
        out_shape = jax.ShapeDtypeStruct((N, D), jnp.float32)
    return pl.pallas_call(
        functools.partial(_tc_layer_body, first, last),
        out_shape=out_shape,
    )(*args)


def kernel(features, edge_index, W1, b1, W2, b2, gamma, beta, Wp, bp):
    src = edge_index[0].astype(jnp.int32)
    dst = edge_index[1].astype(jnp.int32)
    srcs = jnp.pad(src.reshape(NW, EPW),
                   ((0, 0), (0, EPW_PAD - EPW))).reshape(NW, NCHUNK, CHUNK)
    dsts = jnp.pad(dst.reshape(NW, EPW), ((0, 0), (0, EPW_PAD - EPW)),
                   constant_values=N).reshape(NW, NCHUNK, CHUNK)
    zrow = jnp.zeros((RPT, D), jnp.float32)
    ones = jnp.ones((CHUNK, D), jnp.float32)

    h = features
    invd = None
    (dp,) = _deg(dsts, zrow, ones)
    for i in range(3):
        (p,) = _segsum(h, srcs, dsts, zrow)
        if i == 0:
            h, invd = _tc_layer(True, False,
                                (h, p, dp, W1[i], b1[i], W2[i], b2[i],
                                 gamma[i], beta[i]))
        elif i < 2:
            h = _tc_layer(False, False,
                          (h, p, invd, W1[i], b1[i], W2[i], b2[i],
                           gamma[i], beta[i]))
        else:
            return _tc_layer(False, True,
                             (h, p, invd, W1[i], b1[i], W2[i], b2[i],
                              gamma[i], beta[i], Wp, bp))


# speedup vs baseline: 1.4430x; 1.0024x over previous
import functools

import jax
import jax.numpy as jnp
from jax import lax
from jax.experimental import pallas as pl
from jax.experimental.pallas import tpu as pltpu
from jax.experimental.pallas import tpu_sc as plsc

N = 10000
E = 320000
D = 128
NW = 32
EPW = E // NW
CHUNK = 128
NCHUNK = 79
EPW_PAD = NCHUNK * CHUNK
NPAD = 10112
RPT = NPAD // 16


def _segsum_body(h_hbm, srcs_hbm, dsts_hbm, zrow_hbm, out_hbm,
                 src_v, dst_v, rows_v, acc, sem):
    cid = lax.axis_index("c")
    sid = lax.axis_index("s")
    wid = sid * 2 + cid

    r0 = sid * RPT
    pltpu.sync_copy(zrow_hbm, acc.at[pl.ds(r0, RPT)])
    pltpu.sync_copy(srcs_hbm.at[wid], src_v)
    pltpu.sync_copy(dsts_hbm.at[wid], dst_v)
    plsc.subcore_barrier()

    def chunk(j, carry):
        pltpu.async_copy(h_hbm.at[src_v.at[j]], rows_v, sem).wait()
        pltpu.sync_copy(rows_v, acc.at[dst_v.at[j]], add=True)
        return carry

    lax.fori_loop(0, NCHUNK, chunk, 0)
    plsc.subcore_barrier()
    pltpu.sync_copy(acc.at[pl.ds(r0, RPT)], out_hbm.at[cid, pl.ds(r0, RPT)])


def _deg_body(dsts_hbm, zrow_hbm, ones_hbm, outdeg_hbm,
              dst_v, ones_v, accd):
    cid = lax.axis_index("c")
    sid = lax.axis_index("s")
    wid = sid * 2 + cid

    r0 = sid * RPT
    pltpu.sync_copy(zrow_hbm, accd.at[pl.ds(r0, RPT)])
    pltpu.sync_copy(ones_hbm, ones_v)
    pltpu.sync_copy(dsts_hbm.at[wid], dst_v)
    plsc.subcore_barrier()

    def chunk(j, carry):
        pltpu.sync_copy(ones_v, accd.at[dst_v.at[j]], add=True)
        return carry

    lax.fori_loop(0, NCHUNK, chunk, 0)
    plsc.subcore_barrier()
    pltpu.sync_copy(accd.at[pl.ds(r0, RPT)],
                    outdeg_hbm.at[cid, pl.ds(r0, RPT)])


def _sc_mesh():
    return plsc.VectorSubcoreMesh(core_axis_name="c", subcore_axis_name="s")


_segsum = pl.kernel(
    _segsum_body, mesh=_sc_mesh(),
    out_type=[jax.ShapeDtypeStruct((2, NPAD, D), jnp.float32)],
    scratch_types=[
        pltpu.VMEM((NCHUNK, CHUNK), jnp.int32),
        pltpu.VMEM((NCHUNK, CHUNK), jnp.int32),
        pltpu.VMEM((CHUNK, D), jnp.float32),
        pltpu.VMEM_SHARED((NPAD, D), jnp.float32),
        pltpu.SemaphoreType.DMA,
    ])

_deg = pl.kernel(
    _deg_body, mesh=_sc_mesh(),
    out_type=[jax.ShapeDtypeStruct((2, NPAD, D), jnp.float32)],
    scratch_types=[
        pltpu.VMEM((NCHUNK, CHUNK), jnp.int32),
        pltpu.VMEM((CHUNK, D), jnp.float32),
        pltpu.VMEM_SHARED((NPAD, D), jnp.float32),
    ])


def _tc_layer_body(first, last, *refs):
    if first:
        (h_ref, p_ref, dp_ref, w1_ref, b1_ref, w2_ref, b2_ref, g_ref, be_ref,
         hn_ref, invd_ref) = refs
    elif last:
        (h_ref, p_ref, invd_in, w1_ref, b1_ref, w2_ref, b2_ref, g_ref, be_ref,
         wp_ref, bp_ref, out_ref) = refs
    else:
        (h_ref, p_ref, invd_in, w1_ref, b1_ref, w2_ref, b2_ref, g_ref, be_ref,
         hn_ref) = refs

    h = h_ref[...]
    p = p_ref[0, :N, :] + p_ref[1, :N, :]
    if first:
        deg = dp_ref[0, :N, 0:1] + dp_ref[1, :N, 0:1]
        invd = 1.0 / jnp.maximum(deg, 1.0)
        invd_ref[...] = invd
    else:
        invd = invd_in[...]
    r = h + p * invd
    z = jnp.maximum(jnp.dot(r, w1_ref[...],
                            preferred_element_type=jnp.float32) + b1_ref[...],
                    0.0)
    z = jnp.dot(z, w2_ref[...], preferred_element_type=jnp.float32) + b2_ref[...]
    mu = jnp.mean(z, axis=0, keepdims=True)
    zc = z - mu
    var = jnp.mean(zc * zc, axis=0, keepdims=True)
    hn = jnp.maximum(zc * (lax.rsqrt(var + 1e-5) * g_ref[...]) + be_ref[...],
                     0.0)
    if last:
        hg = jnp.mean(hn, axis=0, keepdims=True)
        out_ref[...] = jnp.dot(hg, wp_ref[...],
                               preferred_element_type=jnp.float32) + bp_ref[...]
    else:
        hn_ref[...] = hn


def _tc_layer(first, last, args):
    if first:
        out_shape = [jax.ShapeDtypeStruct((N, D), jnp.float32),
                     jax.ShapeDtypeStruct((N, 1), jnp.float32)]
    elif last:
        out_shape = jax.ShapeDtypeStruct((1, D), jnp.float32)
    else:
        out_shape = jax.ShapeDtypeStruct((N, D), jnp.float32)
    return pl.pallas_call(
        functools.partial(_tc_layer_body, first, last),
        out_shape=out_shape,
    )(*args)


def kernel(features, edge_index, W1, b1, W2, b2, gamma, beta, Wp, bp):
    src = edge_index[0].astype(jnp.int32)
    dst = edge_index[1].astype(jnp.int32)
    srcs = jnp.pad(src.reshape(NW, EPW),
                   ((0, 0), (0, EPW_PAD - EPW))).reshape(NW, NCHUNK, CHUNK)
    pad_dst = jnp.broadcast_to(
        (N + jnp.arange(NW, dtype=jnp.int32))[:, None], (NW, EPW_PAD - EPW))
    dsts = jnp.concatenate([dst.reshape(NW, EPW), pad_dst],
                           axis=1).reshape(NW, NCHUNK, CHUNK)
    zrow = jnp.zeros((RPT, D), jnp.float32)
    ones = jnp.ones((CHUNK, D), jnp.float32)

    h = features
    invd = None
    (dp,) = _deg(dsts, zrow, ones)
    for i in range(3):
        (p,) = _segsum(h, srcs, dsts, zrow)
        if i == 0:
            h, invd = _tc_layer(True, False,
                                (h, p, dp, W1[i], b1[i], W2[i], b2[i],
                                 gamma[i], beta[i]))
        elif i < 2:
            h = _tc_layer(False, False,
                          (h, p, invd, W1[i], b1[i], W2[i], b2[i],
                           gamma[i], beta[i]))
        else:
            return _tc_layer(False, True,
                             (h, p, invd, W1[i], b1[i], W2[i], b2[i],
                              gamma[i], beta[i], Wp, bp))
